# baseline (speedup 1.0000x reference)
import functools

import jax
import jax.numpy as jnp
from jax import lax
from jax.experimental import pallas as pl
from jax.experimental.pallas import tpu as pltpu
from jax.experimental.pallas import tpu_sc as plsc

B = 16384
D = 32
L = 16
NC = 2
NS = 16
NW = NC * NS
BPW = B // NW
CHUNK = 128
NCHUNK = BPW // CHUNK
NGROUP = BPW // L


def _body(user_hbm, item_hbm, eu_hbm, ei_hbm, bu_hbm, bi_hbm, out_hbm,
          uidx, iidx, ru, rv, bu, bi, outv, sem):
    wid = lax.axis_index("s") * NC + lax.axis_index("c")
    base = wid * BPW

    for j in range(NCHUNK):
        pltpu.sync_copy(user_hbm.at[pl.ds(base + j * CHUNK, CHUNK)],
                        uidx.at[j])
        pltpu.sync_copy(item_hbm.at[pl.ds(base + j * CHUNK, CHUNK)],
                        iidx.at[j])

    copies = []
    for j in range(NCHUNK):
        copies.append(pltpu.async_copy(eu_hbm.at[uidx.at[j]], ru.at[j], sem))
        copies.append(pltpu.async_copy(ei_hbm.at[iidx.at[j]], rv.at[j], sem))
        copies.append(pltpu.async_copy(bu_hbm.at[uidx.at[j]], bu.at[j], sem))
        copies.append(pltpu.async_copy(bi_hbm.at[iidx.at[j]], bi.at[j], sem))
    for c in copies:
        c.wait()

    iota = lax.iota(jnp.int32, L)

    def group(g, _):
        row = g * L + iota
        cidx = row >> 7
        pos = row & (CHUNK - 1)
        acc = (plsc.load_gather(bu, [cidx, pos])
               + plsc.load_gather(bi, [cidx, pos])
               + 3.5)
        col = iota
        for _d in range(D):
            uval = plsc.load_gather(ru, [cidx, pos, col])
            vval = plsc.load_gather(rv, [cidx, pos, col])
            acc = acc + uval * vval
            col = (col + 1) & (D - 1)
        acc = jnp.minimum(jnp.maximum(acc, 0.0), 5.0)
        outv[pl.ds(pl.multiple_of(g * L, L), L)] = acc
        return 0

    lax.fori_loop(0, NGROUP, group, 0)

    pltpu.sync_copy(outv, out_hbm.at[pl.ds(base, BPW)])


@jax.jit
def _bpr(user, item, embed_user, embed_item, bias_user, bias_item):
    mesh = plsc.VectorSubcoreMesh(core_axis_name="c", subcore_axis_name="s",
                                  num_cores=NC, num_subcores=NS)
    run = functools.partial(
        pl.kernel,
        out_type=jax.ShapeDtypeStruct((B,), jnp.float32),
        mesh=mesh,
        compiler_params=pltpu.CompilerParams(needs_layout_passes=False, use_tc_tiling_on_sc=False),
        scratch_types=[
            pltpu.VMEM((NCHUNK, CHUNK), jnp.int32),
            pltpu.VMEM((NCHUNK, CHUNK), jnp.int32),
            pltpu.VMEM((NCHUNK, CHUNK, D), jnp.float32),
            pltpu.VMEM((NCHUNK, CHUNK, D), jnp.float32),
            pltpu.VMEM((NCHUNK, CHUNK), jnp.float32),
            pltpu.VMEM((NCHUNK, CHUNK), jnp.float32),
            pltpu.VMEM((BPW,), jnp.float32),
            pltpu.SemaphoreType.DMA,
        ],
    )(_body)
    return run(user, item, embed_user, embed_item,
               bias_user.reshape(-1), bias_item.reshape(-1))


def kernel(user, item, embed_user, embed_item, bias_user, bias_item):
    return _bpr(user, item, embed_user, embed_item, bias_user, bias_item)


# speedup vs baseline: 1.0016x; 1.0016x over previous
import functools

import jax
import jax.numpy as jnp
from jax import lax
from jax.experimental import pallas as pl
from jax.experimental.pallas import tpu as pltpu
from jax.experimental.pallas import tpu_sc as plsc

B = 16384
D = 32
L = 16
NC = 2
NS = 16
NW = NC * NS
BPW = B // NW
CHUNK = 128
NCHUNK = BPW // CHUNK
GPC = CHUNK // L


def _body(user_hbm, item_hbm, eu_hbm, ei_hbm, bu_hbm, bi_hbm, out_hbm,
          uidx, iidx, ru, rv, bu, bi, outv, s0, s1, s2, s3):
    sems = [s0, s1, s2, s3]
    wid = lax.axis_index("s") * NC + lax.axis_index("c")
    base = wid * BPW

    copies = []
    for j in range(NCHUNK):
        pltpu.sync_copy(user_hbm.at[pl.ds(base + j * CHUNK, CHUNK)],
                        uidx.at[j])
        pltpu.sync_copy(item_hbm.at[pl.ds(base + j * CHUNK, CHUNK)],
                        iidx.at[j])
        dst = pl.ds(j * CHUNK, CHUNK)
        copies.append((
            pltpu.async_copy(eu_hbm.at[uidx.at[j]], ru.at[dst], sems[j]),
            pltpu.async_copy(ei_hbm.at[iidx.at[j]], rv.at[dst], sems[j]),
            pltpu.async_copy(bu_hbm.at[uidx.at[j]], bu.at[dst], sems[j]),
            pltpu.async_copy(bi_hbm.at[iidx.at[j]], bi.at[dst], sems[j]),
        ))

    iota = lax.iota(jnp.int32, L)

    for j in range(NCHUNK):
        for c in copies[j]:
            c.wait()

        def group(g, _, j=j):
            row = j * CHUNK + g * L + iota
            accs = [jnp.zeros((L,), jnp.float32) for _ in range(4)]
            col = iota
            for d in range(D):
                uval = plsc.load_gather(ru, [row, col])
                vval = plsc.load_gather(rv, [row, col])
                accs[d & 3] = accs[d & 3] + uval * vval
                col = (col + 1) & (D - 1)
            acc = ((accs[0] + accs[1]) + (accs[2] + accs[3])
                   + plsc.load_gather(bu, [row])
                   + plsc.load_gather(bi, [row])
                   + 3.5)
            acc = jnp.minimum(jnp.maximum(acc, 0.0), 5.0)
            outv[pl.ds(pl.multiple_of(j * CHUNK + g * L, L), L)] = acc
            return 0

        lax.fori_loop(0, GPC, group, 0)

    pltpu.sync_copy(outv, out_hbm.at[pl.ds(base, BPW)])


@jax.jit
def _bpr(user, item, embed_user, embed_item, bias_user, bias_item):
    mesh = plsc.VectorSubcoreMesh(core_axis_name="c", subcore_axis_name="s",
                                  num_cores=NC, num_subcores=NS)
    run = functools.partial(
        pl.kernel,
        out_type=jax.ShapeDtypeStruct((B,), jnp.float32),
        mesh=mesh,
        compiler_params=pltpu.CompilerParams(needs_layout_passes=False,
                                             use_tc_tiling_on_sc=False),
        scratch_types=[
            pltpu.VMEM((NCHUNK, CHUNK), jnp.int32),
            pltpu.VMEM((NCHUNK, CHUNK), jnp.int32),
            pltpu.VMEM((BPW, D), jnp.float32),
            pltpu.VMEM((BPW, D), jnp.float32),
            pltpu.VMEM((BPW,), jnp.float32),
            pltpu.VMEM((BPW,), jnp.float32),
            pltpu.VMEM((BPW,), jnp.float32),
            pltpu.SemaphoreType.DMA,
            pltpu.SemaphoreType.DMA,
            pltpu.SemaphoreType.DMA,
            pltpu.SemaphoreType.DMA,
        ],
    )(_body)
    return run(user, item, embed_user, embed_item,
               bias_user.reshape(-1), bias_item.reshape(-1))


def kernel(user, item, embed_user, embed_item, bias_user, bias_item):
    return _bpr(user, item, embed_user, embed_item, bias_user, bias_item)
